# Initial kernel scaffold; baseline (speedup 1.0000x reference)
import functools

import jax
import jax.numpy as jnp
from jax import lax
from jax.experimental import pallas as pl
from jax.experimental.pallas import tpu as pltpu
from jax.experimental.pallas import tpu_sc as plsc

VOCAB_SIZE = 1000000
EMBEDDING_DIM = 32

NUM_CORES = 2
NUM_SUBCORES = 16
NW = NUM_CORES * NUM_SUBCORES

B_TOTAL = 16384 * 50
BPW = B_TOTAL // NW
CH = 128
NCH = BPW // CH
G = 8
NGRP = NCH // G


def _make_gather():
    mesh = plsc.VectorSubcoreMesh(
        core_axis_name="c", subcore_axis_name="s",
        num_cores=NUM_CORES, num_subcores=NUM_SUBCORES)

    @functools.partial(
        pl.kernel,
        out_type=jax.ShapeDtypeStruct((B_TOTAL, EMBEDDING_DIM), jnp.float32),
        mesh=mesh,
        scratch_types=[
            pltpu.VMEM((NCH, CH), jnp.int32),
            pltpu.VMEM((G * CH, EMBEDDING_DIM), jnp.float32),
            pltpu.SemaphoreType.DMA,
        ],
    )
    def gather_kernel(table_hbm, idx_hbm, out_hbm, idx_v, rows_v, sem):
        wid = lax.axis_index("s") * NUM_CORES + lax.axis_index("c")
        base = wid * BPW
        pltpu.sync_copy(idx_hbm.at[wid], idx_v)

        def group_body(g, _):
            for b in range(G):
                pltpu.async_copy(
                    table_hbm.at[idx_v.at[g * G + b]],
                    rows_v.at[pl.ds(b * CH, CH)],
                    sem,
                )
            for b in range(G):
                pltpu.make_async_copy(
                    table_hbm.at[idx_v.at[g * G + b]],
                    rows_v.at[pl.ds(b * CH, CH)],
                    sem,
                ).wait()
            pltpu.sync_copy(rows_v, out_hbm.at[pl.ds(base + g * G * CH, G * CH)])
            return 0

        lax.fori_loop(0, NGRP, group_body, 0)

    return gather_kernel


_gather = _make_gather()


def kernel(token_ids, embedding_matrix):
    idx = token_ids.astype(jnp.int32).reshape(NW, NCH, CH)
    out = _gather(embedding_matrix, idx)
    return out.reshape(token_ids.shape + (EMBEDDING_DIM,))


# speedup vs baseline: 1.1031x; 1.1031x over previous
import functools

import jax
import jax.numpy as jnp
from jax import lax
from jax.experimental import pallas as pl
from jax.experimental.pallas import tpu as pltpu
from jax.experimental.pallas import tpu_sc as plsc

VOCAB_SIZE = 1000000
EMBEDDING_DIM = 32

NUM_CORES = 2
NUM_SUBCORES = 16
NW = NUM_CORES * NUM_SUBCORES

B_TOTAL = 16384 * 50
BPW = B_TOTAL // NW
CH = 128
NCH = BPW // CH
G = 8
NGRP = NCH // G


def _make_gather():
    mesh = plsc.VectorSubcoreMesh(
        core_axis_name="c", subcore_axis_name="s",
        num_cores=NUM_CORES, num_subcores=NUM_SUBCORES)

    @functools.partial(
        pl.kernel,
        out_type=jax.ShapeDtypeStruct((B_TOTAL, EMBEDDING_DIM), jnp.float32),
        mesh=mesh,
        scratch_types=[
            pltpu.VMEM((NCH, CH), jnp.int32),
            pltpu.VMEM((G * CH, EMBEDDING_DIM), jnp.float32),
            pltpu.SemaphoreType.DMA,
        ],
        compiler_params=pltpu.CompilerParams(use_tc_tiling_on_sc=False),
    )
    def gather_kernel(table_hbm, idx_hbm, out_hbm, idx_v, rows_v, sem):
        wid = lax.axis_index("s") * NUM_CORES + lax.axis_index("c")
        base = wid * BPW
        pltpu.sync_copy(idx_hbm.at[wid], idx_v)

        def group_body(g, _):
            for b in range(G):
                pltpu.async_copy(
                    table_hbm.at[idx_v.at[g * G + b]],
                    rows_v.at[pl.ds(b * CH, CH)],
                    sem,
                )
            for b in range(G):
                pltpu.make_async_copy(
                    table_hbm.at[idx_v.at[g * G + b]],
                    rows_v.at[pl.ds(b * CH, CH)],
                    sem,
                ).wait()
            pltpu.sync_copy(rows_v, out_hbm.at[pl.ds(base + g * G * CH, G * CH)])
            return 0

        lax.fori_loop(0, NGRP, group_body, 0)

    return gather_kernel


_gather = _make_gather()


def kernel(token_ids, embedding_matrix):
    idx = token_ids.astype(jnp.int32).reshape(NW, NCH, CH)
    out = _gather(embedding_matrix, idx)
    return out.reshape(token_ids.shape + (EMBEDDING_DIM,))


# speedup vs baseline: 1.3863x; 1.2567x over previous
import functools

import jax
import jax.numpy as jnp
from jax import lax
from jax.experimental import pallas as pl
from jax.experimental.pallas import tpu as pltpu
from jax.experimental.pallas import tpu_sc as plsc

VOCAB_SIZE = 1000000
D = 32
P = 50
T = 16384

NUM_CORES = 2
NUM_SUBCORES = 16
NW = NUM_CORES * NUM_SUBCORES
TPW = T // NW
CH = 128
NCH = TPW // CH


def _make_gather():
    mesh = plsc.VectorSubcoreMesh(
        core_axis_name="c", subcore_axis_name="s",
        num_cores=NUM_CORES, num_subcores=NUM_SUBCORES)

    @functools.partial(
        pl.kernel,
        out_type=jax.ShapeDtypeStruct((P, D, T), jnp.float32),
        mesh=mesh,
        scratch_types=[
            pltpu.VMEM((P, TPW), jnp.int32),
            pltpu.VMEM((TPW, D), jnp.float32),
            pltpu.VMEM((D, TPW), jnp.float32),
            pltpu.SemaphoreType.DMA,
        ],
        compiler_params=pltpu.CompilerParams(
            use_tc_tiling_on_sc=False, needs_layout_passes=False),
    )
    def gather_kernel(table_hbm, ids_hbm, out_hbm, ids_v, rows_v, tr_v, sem):
        wid = lax.axis_index("s") * NUM_CORES + lax.axis_index("c")
        t0 = wid * TPW
        pltpu.sync_copy(ids_hbm.at[:, pl.ds(t0, TPW)], ids_v)

        lane = lax.iota(jnp.int32, 16)

        def pos_body(p, _):
            for c in range(NCH):
                pltpu.async_copy(
                    table_hbm.at[ids_v.at[p, pl.ds(c * CH, CH)]],
                    rows_v.at[pl.ds(c * CH, CH)],
                    sem,
                )
            for c in range(NCH):
                pltpu.make_async_copy(
                    table_hbm.at[ids_v.at[p, pl.ds(c * CH, CH)]],
                    rows_v.at[pl.ds(c * CH, CH)],
                    sem,
                ).wait()

            def tr_body(t16, _):
                row_idx = t16 * 16 + lane
                for f in range(D):
                    col_idx = jnp.full((16,), f, dtype=jnp.int32)
                    vals = plsc.load_gather(rows_v, [row_idx, col_idx])
                    tr_v[f, pl.ds(t16 * 16, 16)] = vals
                return 0

            lax.fori_loop(0, TPW // 16, tr_body, 0)
            pltpu.sync_copy(tr_v, out_hbm.at[p, :, pl.ds(t0, TPW)])
            return 0

        lax.fori_loop(0, P, pos_body, 0)

    return gather_kernel


_gather = _make_gather()


def kernel(token_ids, embedding_matrix):
    ids_t = token_ids.astype(jnp.int32).T
    out = _gather(embedding_matrix, ids_t)
    return out.transpose(2, 0, 1)


# speedup vs baseline: 1.9480x; 1.4052x over previous
import functools

import jax
import jax.numpy as jnp
from jax import lax
from jax.experimental import pallas as pl
from jax.experimental.pallas import tpu as pltpu
from jax.experimental.pallas import tpu_sc as plsc

VOCAB_SIZE = 1000000
D = 32
P = 50
T = 16384

NUM_CORES = 2
NUM_SUBCORES = 16
NW = NUM_CORES * NUM_SUBCORES
TPW = T // NW
CH = 128
NCH = TPW // CH


def _make_gather():
    mesh = plsc.VectorSubcoreMesh(
        core_axis_name="c", subcore_axis_name="s",
        num_cores=NUM_CORES, num_subcores=NUM_SUBCORES)

    @functools.partial(
        pl.kernel,
        out_type=jax.ShapeDtypeStruct((P, T, D), jnp.float32),
        mesh=mesh,
        scratch_types=[
            pltpu.VMEM((P, TPW), jnp.int32),
            pltpu.VMEM((2, TPW, D), jnp.float32),
            pltpu.SemaphoreType.DMA,
            pltpu.SemaphoreType.DMA,
        ],
        compiler_params=pltpu.CompilerParams(
            use_tc_tiling_on_sc=False, needs_layout_passes=False),
    )
    def gather_kernel(table_hbm, ids_hbm, out_hbm, ids_v, rows_v, semA, semB):
        wid = lax.axis_index("s") * NUM_CORES + lax.axis_index("c")
        t0 = wid * TPW
        pltpu.sync_copy(ids_hbm.at[:, pl.ds(t0, TPW)], ids_v)

        def fire(p, b, sem):
            for c in range(NCH):
                pltpu.async_copy(
                    table_hbm.at[ids_v.at[p, pl.ds(c * CH, CH)]],
                    rows_v.at[b, pl.ds(c * CH, CH)],
                    sem,
                )

        def drain(p, b, sem):
            for c in range(NCH):
                pltpu.make_async_copy(
                    table_hbm.at[ids_v.at[p, pl.ds(c * CH, CH)]],
                    rows_v.at[b, pl.ds(c * CH, CH)],
                    sem,
                ).wait()

        def store(p, b):
            pltpu.sync_copy(rows_v.at[b], out_hbm.at[p, pl.ds(t0, TPW)])

        fire(0, 0, semA)

        def pair_body(i, _):
            p0 = 2 * i
            fire(p0 + 1, 1, semB)
            drain(p0, 0, semA)
            store(p0, 0)
            fire(p0 + 2, 0, semA)
            drain(p0 + 1, 1, semB)
            store(p0 + 1, 1)
            return 0

        lax.fori_loop(0, P // 2 - 1, pair_body, 0)
        fire(P - 1, 1, semB)
        drain(P - 2, 0, semA)
        store(P - 2, 0)
        drain(P - 1, 1, semB)
        store(P - 1, 1)

    return gather_kernel


_gather = _make_gather()


def kernel(token_ids, embedding_matrix):
    ids_t = token_ids.astype(jnp.int32).T
    out = _gather(embedding_matrix, ids_t)
    return out.transpose(1, 0, 2)


# speedup vs baseline: 2.3915x; 1.2277x over previous
import functools

import jax
import jax.numpy as jnp
from jax import lax
from jax.experimental import pallas as pl
from jax.experimental.pallas import tpu as pltpu
from jax.experimental.pallas import tpu_sc as plsc

VOCAB_SIZE = 1000000
D = 32
P = 50
T = 16384

NUM_CORES = 2
NUM_SUBCORES = 16
NW = NUM_CORES * NUM_SUBCORES
TPW = T // NW
CH = 128
NCH = TPW // CH

BLK = 4 * TPW
NVB = (VOCAB_SIZE + BLK - 1) // BLK
VROWS = NVB * BLK
NTB = T // BLK


def _make_gather():
    mesh = plsc.VectorSubcoreMesh(
        core_axis_name="c", subcore_axis_name="s",
        num_cores=NUM_CORES, num_subcores=NUM_SUBCORES)

    @functools.partial(
        pl.kernel,
        out_type=jax.ShapeDtypeStruct((P * NTB, TPW, 4, D), jnp.float32),
        mesh=mesh,
        scratch_types=[
            pltpu.VMEM((P, TPW), jnp.int32),
            pltpu.VMEM((2, TPW, D), jnp.float32),
            pltpu.SemaphoreType.DMA,
            pltpu.SemaphoreType.DMA,
        ],
        compiler_params=pltpu.CompilerParams(
            use_tc_tiling_on_sc=False, needs_layout_passes=False),
    )
    def gather_kernel(table_hbm, ids_hbm, out_hbm, ids_v, rows_v, semA, semB):
        wid = lax.axis_index("s") * NUM_CORES + lax.axis_index("c")
        i_blk = wid // 4
        q = wid % 4
        t0 = wid * TPW
        pltpu.sync_copy(ids_hbm.at[:, pl.ds(t0, TPW)], ids_v)

        def xform_body(k, _):
            pp = k // (TPW // 16)
            off = (k % (TPW // 16)) * 16
            v = ids_v[pp, pl.ds(off, 16)]
            r = ((v >> 11) << 11) + ((v & 511) << 2) + ((v >> 9) & 3)
            ids_v[pp, pl.ds(off, 16)] = r
            return 0

        lax.fori_loop(0, P * TPW // 16, xform_body, 0)

        def fire(p, b, sem):
            for c in range(NCH):
                pltpu.async_copy(
                    table_hbm.at[ids_v.at[p, pl.ds(c * CH, CH)]],
                    rows_v.at[b, pl.ds(c * CH, CH)],
                    sem,
                )

        def drain(p, b, sem):
            for c in range(NCH):
                pltpu.make_async_copy(
                    table_hbm.at[ids_v.at[p, pl.ds(c * CH, CH)]],
                    rows_v.at[b, pl.ds(c * CH, CH)],
                    sem,
                ).wait()

        def store(p, b):
            pltpu.sync_copy(
                rows_v.at[b], out_hbm.at[p * NTB + i_blk, :, q, :])

        fire(0, 0, semA)

        def pair_body(i, _):
            p0 = 2 * i
            fire(p0 + 1, 1, semB)
            drain(p0, 0, semA)
            store(p0, 0)
            fire(p0 + 2, 0, semA)
            drain(p0 + 1, 1, semB)
            store(p0 + 1, 1)
            return 0

        lax.fori_loop(0, P // 2 - 1, pair_body, 0)
        fire(P - 1, 1, semB)
        drain(P - 2, 0, semA)
        store(P - 2, 0)
        drain(P - 1, 1, semB)
        store(P - 1, 1)

    return gather_kernel


_gather = _make_gather()


def _table_body(x_ref, o_ref):
    y = x_ref[...].T
    o_ref[...] = jnp.concatenate(
        [y[qq * TPW:(qq + 1) * TPW, :] for qq in range(4)], axis=1)

_table_ic = pl.pallas_call(
    _table_body,
    grid=(NVB,),
    in_specs=[pl.BlockSpec((D, BLK), lambda i: (0, i))],
    out_specs=pl.BlockSpec((TPW, 4 * D), lambda i: (i, 0)),
    out_shape=jax.ShapeDtypeStruct((NVB * TPW, 4 * D), jnp.float32),
)


def _out_body(x_ref, o_ref):
    y = x_ref[...].T
    o_ref[0] = jnp.concatenate(
        [y[qq * D:(qq + 1) * D, :] for qq in range(4)], axis=1)

_out_final = pl.pallas_call(
    _out_body,
    grid=(P, NTB),
    in_specs=[pl.BlockSpec((TPW, 4 * D), lambda p, i: (p * NTB + i, 0))],
    out_specs=pl.BlockSpec((1, D, BLK), lambda p, i: (p, 0, i)),
    out_shape=jax.ShapeDtypeStruct((P, D, T), jnp.float32),
)


def kernel(token_ids, embedding_matrix):
    ids_t = token_ids.astype(jnp.int32).T
    table_ic = _table_ic(embedding_matrix.T)
    rows = _gather(table_ic.reshape(VROWS, D), ids_t)
    out3 = _out_final(rows.reshape(P * NTB * TPW, 4 * D))
    return out3.transpose(2, 0, 1)


# speedup vs baseline: 2.7058x; 1.1314x over previous
import functools

import jax
import jax.numpy as jnp
from jax import lax
from jax.experimental import pallas as pl
from jax.experimental.pallas import tpu as pltpu
from jax.experimental.pallas import tpu_sc as plsc

VOCAB_SIZE = 1000000
D = 32
P = 50
T = 16384

NUM_CORES = 2
NUM_SUBCORES = 16
NW = NUM_CORES * NUM_SUBCORES
TPW = T // NW
CH = 128
NCH = TPW // CH

BLK = 4 * TPW
NVB = (VOCAB_SIZE + BLK - 1) // BLK
VROWS = NVB * BLK
NTB = T // BLK


def _make_gather():
    mesh = plsc.VectorSubcoreMesh(
        core_axis_name="c", subcore_axis_name="s",
        num_cores=NUM_CORES, num_subcores=NUM_SUBCORES)

    @functools.partial(
        pl.kernel,
        out_type=jax.ShapeDtypeStruct((P * NTB, TPW, 4, D), jnp.float32),
        mesh=mesh,
        scratch_types=[
            pltpu.VMEM((P, TPW), jnp.int32),
            pltpu.VMEM((2, TPW, D), jnp.float32),
            pltpu.SemaphoreType.DMA,
            pltpu.SemaphoreType.DMA,
        ],
        compiler_params=pltpu.CompilerParams(
            use_tc_tiling_on_sc=False, needs_layout_passes=False),
    )
    def gather_kernel(table_hbm, ids_hbm, out_hbm, ids_v, rows_v, semA, semB):
        wid = lax.axis_index("s") * NUM_CORES + lax.axis_index("c")
        i_blk = wid // 4
        q = wid % 4
        t0 = wid * TPW
        pltpu.sync_copy(ids_hbm.at[:, pl.ds(t0, TPW)], ids_v)

        def xform_body(k, _):
            pp = k // (TPW // 16)
            off = (k % (TPW // 16)) * 16
            v = ids_v[pp, pl.ds(off, 16)]
            r = ((v >> 11) << 11) + ((v & 511) << 2) + ((v >> 9) & 3)
            ids_v[pp, pl.ds(off, 16)] = r
            return 0

        lax.fori_loop(0, P * TPW // 16, xform_body, 0)

        def fire(p, b, sem):
            for c in range(NCH):
                pltpu.async_copy(
                    table_hbm.at[ids_v.at[p, pl.ds(c * CH, CH)]],
                    rows_v.at[b, pl.ds(c * CH, CH)],
                    sem,
                )

        def drain(p, b, sem):
            for c in range(NCH):
                pltpu.make_async_copy(
                    table_hbm.at[ids_v.at[p, pl.ds(c * CH, CH)]],
                    rows_v.at[b, pl.ds(c * CH, CH)],
                    sem,
                ).wait()

        def store(p, b):
            pltpu.sync_copy(
                rows_v.at[b], out_hbm.at[p * NTB + i_blk, :, q, :])

        fire(0, 0, semA)

        def pair_body(i, _):
            p0 = 2 * i
            fire(p0 + 1, 1, semB)
            drain(p0, 0, semA)
            store(p0, 0)
            fire(p0 + 2, 0, semA)
            drain(p0 + 1, 1, semB)
            store(p0 + 1, 1)
            return 0

        lax.fori_loop(0, P // 2 - 1, pair_body, 0)
        fire(P - 1, 1, semB)
        drain(P - 2, 0, semA)
        store(P - 2, 0)
        drain(P - 1, 1, semB)
        store(P - 1, 1)

    return gather_kernel


_gather = _make_gather()


def _table_body(x_ref, o_ref):
    z = jnp.concatenate(
        [x_ref[:, qq * TPW:(qq + 1) * TPW] for qq in range(4)], axis=0)
    o_ref[...] = z.T

_table_ic = pl.pallas_call(
    _table_body,
    grid=(NVB,),
    in_specs=[pl.BlockSpec((D, BLK), lambda i: (0, i))],
    out_specs=pl.BlockSpec((TPW, 4 * D), lambda i: (i, 0)),
    out_shape=jax.ShapeDtypeStruct((NVB * TPW, 4 * D), jnp.float32),
)


def _out_body(x_ref, o_ref):
    y = x_ref[...].T
    o_ref[0] = jnp.concatenate(
        [y[qq * D:(qq + 1) * D, :] for qq in range(4)], axis=1)

_out_final = pl.pallas_call(
    _out_body,
    grid=(P, NTB),
    in_specs=[pl.BlockSpec((TPW, 4 * D), lambda p, i: (p * NTB + i, 0))],
    out_specs=pl.BlockSpec((1, D, BLK), lambda p, i: (p, 0, i)),
    out_shape=jax.ShapeDtypeStruct((P, D, T), jnp.float32),
)


def kernel(token_ids, embedding_matrix):
    ids_t = token_ids.astype(jnp.int32).T
    table_ic = _table_ic(embedding_matrix.T)
    rows = _gather(table_ic.reshape(VROWS, D), ids_t)
    out3 = _out_final(rows.reshape(P * NTB * TPW, 4 * D))
    return out3.transpose(2, 0, 1)


# speedup vs baseline: 6.4784x; 2.3943x over previous
import functools

import jax
import jax.numpy as jnp
from jax import lax
from jax.experimental import pallas as pl
from jax.experimental.pallas import tpu as pltpu
from jax.experimental.pallas import tpu_sc as plsc

VOCAB_SIZE = 1000000
D = 32
P = 50
T = 16384

NUM_CORES = 2
NUM_SUBCORES = 16
NW = NUM_CORES * NUM_SUBCORES
TPW = T // NW
CH = 128
NCH = TPW // CH

BLK = 4 * TPW
_MB = 8
NVB = -(-VOCAB_SIZE // (_MB * BLK)) * _MB
VROWS = NVB * BLK
NTB = T // BLK


def _make_gather():
    mesh = plsc.VectorSubcoreMesh(
        core_axis_name="c", subcore_axis_name="s",
        num_cores=NUM_CORES, num_subcores=NUM_SUBCORES)

    @functools.partial(
        pl.kernel,
        out_type=jax.ShapeDtypeStruct((P * NTB, TPW, 4, D), jnp.float32),
        mesh=mesh,
        scratch_types=[
            pltpu.VMEM((P, TPW), jnp.int32),
            pltpu.VMEM((2, TPW, D), jnp.float32),
            pltpu.SemaphoreType.DMA,
            pltpu.SemaphoreType.DMA,
        ],
        compiler_params=pltpu.CompilerParams(
            use_tc_tiling_on_sc=False, needs_layout_passes=False),
    )
    def gather_kernel(table_hbm, ids_hbm, out_hbm, ids_v, rows_v, semA, semB):
        wid = lax.axis_index("s") * NUM_CORES + lax.axis_index("c")
        i_blk = wid // 4
        q = wid % 4
        t0 = wid * TPW
        pltpu.sync_copy(ids_hbm.at[:, pl.ds(t0, TPW)], ids_v)

        def xform_body(k, _):
            pp = k // (TPW // 16)
            off = (k % (TPW // 16)) * 16
            v = ids_v[pp, pl.ds(off, 16)]
            r = ((v >> 11) << 11) + ((v & 511) << 2) + ((v >> 9) & 3)
            ids_v[pp, pl.ds(off, 16)] = r
            return 0

        lax.fori_loop(0, P * TPW // 16, xform_body, 0)

        def fire(p, b, sem):
            for c in range(NCH):
                pltpu.async_copy(
                    table_hbm.at[ids_v.at[p, pl.ds(c * CH, CH)]],
                    rows_v.at[b, pl.ds(c * CH, CH)],
                    sem,
                )

        def drain(p, b, sem):
            for c in range(NCH):
                pltpu.make_async_copy(
                    table_hbm.at[ids_v.at[p, pl.ds(c * CH, CH)]],
                    rows_v.at[b, pl.ds(c * CH, CH)],
                    sem,
                ).wait()

        def store(p, b):
            pltpu.sync_copy(
                rows_v.at[b], out_hbm.at[p * NTB + i_blk, :, q, :])

        fire(0, 0, semA)

        def pair_body(i, _):
            p0 = 2 * i
            fire(p0 + 1, 1, semB)
            drain(p0, 0, semA)
            store(p0, 0)
            fire(p0 + 2, 0, semA)
            drain(p0 + 1, 1, semB)
            store(p0 + 1, 1)
            return 0

        lax.fori_loop(0, P // 2 - 1, pair_body, 0)
        fire(P - 1, 1, semB)
        drain(P - 2, 0, semA)
        store(P - 2, 0)
        drain(P - 1, 1, semB)
        store(P - 1, 1)

    return gather_kernel


_gather = _make_gather()


MB = _MB

def _table_body(x_ref, o_ref):
    for j in range(MB):
        z = jnp.concatenate(
            [x_ref[:, j * BLK + qq * TPW:j * BLK + (qq + 1) * TPW]
             for qq in range(4)], axis=0)
        o_ref[pl.ds(j * TPW, TPW), :] = z.T

_table_ic = pl.pallas_call(
    _table_body,
    grid=(NVB // MB,),
    in_specs=[pl.BlockSpec((D, MB * BLK), lambda i: (0, i))],
    out_specs=pl.BlockSpec((MB * TPW, 4 * D), lambda i: (i, 0)),
    out_shape=jax.ShapeDtypeStruct((NVB * TPW, 4 * D), jnp.float32),
)


def _out_body(x_ref, o_ref):
    pieces = []
    for j in range(NTB):
        y = x_ref[pl.ds(j * TPW, TPW), :].T
        pieces.extend(y[qq * D:(qq + 1) * D, :] for qq in range(4))
    o_ref[0] = jnp.concatenate(pieces, axis=1)

_out_final = pl.pallas_call(
    _out_body,
    grid=(P,),
    in_specs=[pl.BlockSpec((NTB * TPW, 4 * D), lambda p: (p, 0))],
    out_specs=pl.BlockSpec((1, D, T), lambda p: (p, 0, 0)),
    out_shape=jax.ShapeDtypeStruct((P, D, T), jnp.float32),
)


def kernel(token_ids, embedding_matrix):
    ids_t = token_ids.astype(jnp.int32).T
    table_ic = _table_ic(embedding_matrix.T)
    rows = _gather(table_ic.reshape(VROWS, D), ids_t)
    out3 = _out_final(rows.reshape(P * NTB * TPW, 4 * D))
    return out3.transpose(2, 0, 1)


# speedup vs baseline: 7.1911x; 1.1100x over previous
import functools

import jax
import jax.numpy as jnp
from jax import lax
from jax.experimental import pallas as pl
from jax.experimental.pallas import tpu as pltpu
from jax.experimental.pallas import tpu_sc as plsc

VOCAB_SIZE = 1000000
D = 32
P = 50
T = 16384

NUM_CORES = 2
NUM_SUBCORES = 16
NW = NUM_CORES * NUM_SUBCORES
TPW = T // NW
CH = 128
NCH = TPW // CH

BLK = 4 * TPW
_MB = 16
NVB = -(-VOCAB_SIZE // (_MB * BLK)) * _MB
VROWS = NVB * BLK
NTB = T // BLK


def _make_gather():
    mesh = plsc.VectorSubcoreMesh(
        core_axis_name="c", subcore_axis_name="s",
        num_cores=NUM_CORES, num_subcores=NUM_SUBCORES)

    @functools.partial(
        pl.kernel,
        out_type=jax.ShapeDtypeStruct((P * NTB, TPW, 4, D), jnp.float32),
        mesh=mesh,
        scratch_types=[
            pltpu.VMEM((P, TPW), jnp.int32),
            pltpu.VMEM((2, TPW, D), jnp.float32),
            pltpu.SemaphoreType.DMA,
            pltpu.SemaphoreType.DMA,
        ],
        compiler_params=pltpu.CompilerParams(
            use_tc_tiling_on_sc=False, needs_layout_passes=False),
    )
    def gather_kernel(table_hbm, ids_hbm, out_hbm, ids_v, rows_v, semA, semB):
        wid = lax.axis_index("s") * NUM_CORES + lax.axis_index("c")
        i_blk = wid // 4
        q = wid % 4
        t0 = wid * TPW
        pltpu.sync_copy(ids_hbm.at[:, pl.ds(t0, TPW)], ids_v)

        def xform_body(k, _):
            pp = k // (TPW // 16)
            off = (k % (TPW // 16)) * 16
            v = ids_v[pp, pl.ds(off, 16)]
            r = ((v >> 11) << 11) + ((v & 511) << 2) + ((v >> 9) & 3)
            ids_v[pp, pl.ds(off, 16)] = r
            return 0

        lax.fori_loop(0, P * TPW // 16, xform_body, 0)

        def fire(p, b, sem):
            for c in range(NCH):
                pltpu.async_copy(
                    table_hbm.at[ids_v.at[p, pl.ds(c * CH, CH)]],
                    rows_v.at[b, pl.ds(c * CH, CH)],
                    sem,
                )

        def drain(p, b, sem):
            for c in range(NCH):
                pltpu.make_async_copy(
                    table_hbm.at[ids_v.at[p, pl.ds(c * CH, CH)]],
                    rows_v.at[b, pl.ds(c * CH, CH)],
                    sem,
                ).wait()

        def store(p, b):
            pltpu.sync_copy(
                rows_v.at[b], out_hbm.at[p * NTB + i_blk, :, q, :])

        fire(0, 0, semA)

        def pair_body(i, _):
            p0 = 2 * i
            fire(p0 + 1, 1, semB)
            drain(p0, 0, semA)
            store(p0, 0)
            fire(p0 + 2, 0, semA)
            drain(p0 + 1, 1, semB)
            store(p0 + 1, 1)
            return 0

        lax.fori_loop(0, P // 2 - 1, pair_body, 0)
        fire(P - 1, 1, semB)
        drain(P - 2, 0, semA)
        store(P - 2, 0)
        drain(P - 1, 1, semB)
        store(P - 1, 1)

    return gather_kernel


_gather = _make_gather()


MB = _MB

def _table_body(x_ref, o_ref):
    for j in range(MB):
        z = jnp.concatenate(
            [x_ref[:, j * BLK + qq * TPW:j * BLK + (qq + 1) * TPW]
             for qq in range(4)], axis=0)
        o_ref[pl.ds(j * TPW, TPW), :] = z.T

_table_ic = pl.pallas_call(
    _table_body,
    grid=(NVB // MB,),
    in_specs=[pl.BlockSpec((D, MB * BLK), lambda i: (0, i))],
    out_specs=pl.BlockSpec((MB * TPW, 4 * D), lambda i: (i, 0)),
    out_shape=jax.ShapeDtypeStruct((NVB * TPW, 4 * D), jnp.float32),
)


def _out_body(x_ref, o_ref):
    for pp in range(2):
        pieces = []
        for j in range(NTB):
            y = x_ref[pl.ds((pp * NTB + j) * TPW, TPW), :].T
            pieces.extend(y[qq * D:(qq + 1) * D, :] for qq in range(4))
        o_ref[pp] = jnp.concatenate(pieces, axis=1)

_out_final = pl.pallas_call(
    _out_body,
    grid=(P // 2,),
    in_specs=[pl.BlockSpec((2 * NTB * TPW, 4 * D), lambda p: (p, 0))],
    out_specs=pl.BlockSpec((2, D, T), lambda p: (p, 0, 0)),
    out_shape=jax.ShapeDtypeStruct((P, D, T), jnp.float32),
)


def kernel(token_ids, embedding_matrix):
    ids_t = token_ids.astype(jnp.int32).T
    table_ic = _table_ic(embedding_matrix.T)
    rows = _gather(table_ic.reshape(VROWS, D), ids_t)
    out3 = _out_final(rows.reshape(P * NTB * TPW, 4 * D))
    return out3.transpose(2, 0, 1)


# speedup vs baseline: 7.2499x; 1.0082x over previous
import functools

import jax
import jax.numpy as jnp
from jax import lax
from jax.experimental import pallas as pl
from jax.experimental.pallas import tpu as pltpu
from jax.experimental.pallas import tpu_sc as plsc

VOCAB_SIZE = 1000000
D = 32
P = 50
T = 16384

NUM_CORES = 2
NUM_SUBCORES = 16
NW = NUM_CORES * NUM_SUBCORES
TPW = T // NW
CH = 128
NCH = TPW // CH

BLK = 4 * TPW
_MB = 32
NVB = -(-VOCAB_SIZE // (_MB * BLK)) * _MB
VROWS = NVB * BLK
NTB = T // BLK


def _make_gather():
    mesh = plsc.VectorSubcoreMesh(
        core_axis_name="c", subcore_axis_name="s",
        num_cores=NUM_CORES, num_subcores=NUM_SUBCORES)

    @functools.partial(
        pl.kernel,
        out_type=jax.ShapeDtypeStruct((P * NTB, TPW, 4, D), jnp.float32),
        mesh=mesh,
        scratch_types=[
            pltpu.VMEM((P, TPW), jnp.int32),
            pltpu.VMEM((2, TPW, D), jnp.float32),
            pltpu.SemaphoreType.DMA,
            pltpu.SemaphoreType.DMA,
        ],
        compiler_params=pltpu.CompilerParams(
            use_tc_tiling_on_sc=False, needs_layout_passes=False),
    )
    def gather_kernel(table_hbm, ids_hbm, out_hbm, ids_v, rows_v, semA, semB):
        wid = lax.axis_index("s") * NUM_CORES + lax.axis_index("c")
        i_blk = wid // 4
        q = wid % 4
        t0 = wid * TPW
        pltpu.sync_copy(ids_hbm.at[:, pl.ds(t0, TPW)], ids_v)

        def xform_body(k, _):
            pp = k // (TPW // 16)
            off = (k % (TPW // 16)) * 16
            v = ids_v[pp, pl.ds(off, 16)]
            r = ((v >> 11) << 11) + ((v & 511) << 2) + ((v >> 9) & 3)
            ids_v[pp, pl.ds(off, 16)] = r
            return 0

        lax.fori_loop(0, P * TPW // 16, xform_body, 0)

        def fire(p, b, sem):
            for c in range(NCH):
                pltpu.async_copy(
                    table_hbm.at[ids_v.at[p, pl.ds(c * CH, CH)]],
                    rows_v.at[b, pl.ds(c * CH, CH)],
                    sem,
                )

        def drain(p, b, sem):
            for c in range(NCH):
                pltpu.make_async_copy(
                    table_hbm.at[ids_v.at[p, pl.ds(c * CH, CH)]],
                    rows_v.at[b, pl.ds(c * CH, CH)],
                    sem,
                ).wait()

        def store(p, b):
            pltpu.sync_copy(
                rows_v.at[b], out_hbm.at[p * NTB + i_blk, :, q, :])

        fire(0, 0, semA)

        def pair_body(i, _):
            p0 = 2 * i
            fire(p0 + 1, 1, semB)
            drain(p0, 0, semA)
            store(p0, 0)
            fire(p0 + 2, 0, semA)
            drain(p0 + 1, 1, semB)
            store(p0 + 1, 1)
            return 0

        lax.fori_loop(0, P // 2 - 1, pair_body, 0)
        fire(P - 1, 1, semB)
        drain(P - 2, 0, semA)
        store(P - 2, 0)
        drain(P - 1, 1, semB)
        store(P - 1, 1)

    return gather_kernel


_gather = _make_gather()


MB = _MB

def _table_body(x_ref, o_ref):
    for j in range(MB):
        z = jnp.concatenate(
            [x_ref[:, j * BLK + qq * TPW:j * BLK + (qq + 1) * TPW]
             for qq in range(4)], axis=0)
        o_ref[pl.ds(j * TPW, TPW), :] = z.T

_table_ic = pl.pallas_call(
    _table_body,
    grid=(NVB // MB,),
    in_specs=[pl.BlockSpec((D, MB * BLK), lambda i: (0, i))],
    out_specs=pl.BlockSpec((MB * TPW, 4 * D), lambda i: (i, 0)),
    out_shape=jax.ShapeDtypeStruct((NVB * TPW, 4 * D), jnp.float32),
)


_PP = 5

def _out_body(x_ref, o_ref):
    for pp in range(_PP):
        pieces = []
        for j in range(NTB):
            y = x_ref[pl.ds((pp * NTB + j) * TPW, TPW), :].T
            pieces.extend(y[qq * D:(qq + 1) * D, :] for qq in range(4))
        o_ref[pp] = jnp.concatenate(pieces, axis=1)

_out_final = pl.pallas_call(
    _out_body,
    grid=(P // _PP,),
    in_specs=[pl.BlockSpec((_PP * NTB * TPW, 4 * D), lambda p: (p, 0))],
    out_specs=pl.BlockSpec((_PP, D, T), lambda p: (p, 0, 0)),
    out_shape=jax.ShapeDtypeStruct((P, D, T), jnp.float32),
)


def kernel(token_ids, embedding_matrix):
    ids_t = token_ids.astype(jnp.int32).T
    table_ic = _table_ic(embedding_matrix.T)
    rows = _gather(table_ic.reshape(VROWS, D), ids_t)
    out3 = _out_final(rows.reshape(P * NTB * TPW, 4 * D))
    return out3.transpose(2, 0, 1)
